# baseline (speedup 1.0000x reference)
import jax
import jax.numpy as jnp
from jax.experimental import pallas as pl
from jax.experimental.pallas import tpu as pltpu
from jax.experimental.pallas import tpu_sc as plsc

_LANES = 16


def kernel(x, emb_weight):
    seq_len = x.shape[1]
    max_seq, dim = emb_weight.shape
    assert seq_len <= max_seq
    scale = dim ** (-0.5)

    block_rows = 4
    grid = (seq_len // block_rows,)
    vector_mesh = plsc.VectorSubcoreMesh(
        core_axis_name="core", subcore_axis_name="subcore"
    )

    @pl.kernel(
        out_type=jax.ShapeDtypeStruct((seq_len, dim), emb_weight.dtype),
        mesh=vector_mesh,
        scratch_types=[],
    )
    def sc_scale_copy(w_hbm, o_hbm):
        def body(in_vmem, out_vmem):
            @pl.loop(0, block_rows)
            def _(r):
                @pl.loop(0, dim, step=_LANES)
                def _(c):
                    slc = (pl.ds(r, 1), pl.ds(c, _LANES))
                    out_vmem.at[*slc][...] = in_vmem.at[*slc][...] * scale

        pltpu.emit_pipeline(
            body,
            grid=grid,
            in_specs=[pl.BlockSpec((block_rows, dim), lambda i: (i, 0))],
            out_specs=[pl.BlockSpec((block_rows, dim), lambda i: (i, 0))],
            core_axis_name=("core", "subcore"),
            dimension_semantics=(pltpu.PARALLEL,),
        )(w_hbm, o_hbm)

    return sc_scale_copy(emb_weight)


# speedup vs baseline: 1.1403x; 1.1403x over previous
import jax
import jax.numpy as jnp
from jax.experimental import pallas as pl
from jax.experimental.pallas import tpu as pltpu
from jax.experimental.pallas import tpu_sc as plsc

_LANES = 16


def kernel(x, emb_weight):
    seq_len = x.shape[1]
    max_seq, dim = emb_weight.shape
    assert seq_len <= max_seq
    scale = dim ** (-0.5)

    block_rows = 4
    grid = (seq_len // block_rows,)
    vector_mesh = plsc.VectorSubcoreMesh(
        core_axis_name="core", subcore_axis_name="subcore"
    )

    @pl.kernel(
        out_type=jax.ShapeDtypeStruct((seq_len, dim), emb_weight.dtype),
        mesh=vector_mesh,
        scratch_types=[],
    )
    def sc_scale_copy(w_hbm, o_hbm):
        def body(in_vmem, out_vmem):
            out_vmem[...] = in_vmem[...] * scale

        pltpu.emit_pipeline(
            body,
            grid=grid,
            in_specs=[pl.BlockSpec((block_rows, dim), lambda i: (i, 0))],
            out_specs=[pl.BlockSpec((block_rows, dim), lambda i: (i, 0))],
            core_axis_name=("core", "subcore"),
            dimension_semantics=(pltpu.PARALLEL,),
        )(w_hbm, o_hbm)

    return sc_scale_copy(emb_weight)


# speedup vs baseline: 4.5251x; 3.9682x over previous
import jax
import jax.numpy as jnp
from jax.experimental import pallas as pl
from jax.experimental.pallas import tpu as pltpu


def _scale_copy_block(w_ref, o_ref, *, scale):
    o_ref[...] = w_ref[...] * scale


def kernel(x, emb_weight):
    seq_len = x.shape[1]
    max_seq, dim = emb_weight.shape
    assert seq_len <= max_seq
    scale = dim ** (-0.5)
    block_rows = 896
    grid = (pl.cdiv(seq_len, block_rows),)
    import functools
    return pl.pallas_call(
        functools.partial(_scale_copy_block, scale=scale),
        grid=grid,
        in_specs=[pl.BlockSpec((block_rows, dim), lambda i: (i, 0))],
        out_specs=pl.BlockSpec((block_rows, dim), lambda i: (i, 0)),
        out_shape=jax.ShapeDtypeStruct((seq_len, dim), emb_weight.dtype),
        compiler_params=pltpu.CompilerParams(
            vmem_limit_bytes=100 * 1024 * 1024,
        ),
    )(emb_weight)
